# Initial kernel scaffold; baseline (speedup 1.0000x reference)
import jax
import jax.numpy as jnp
from jax.experimental import pallas as pl


def kernel(x, edge_index, W1, b1, W2, b2):
    raise NotImplementedError("write your pallas kernel here")


# speedup vs baseline: 9.5867x; 9.5867x over previous
import functools

import jax
import jax.numpy as jnp
from jax import lax
from jax.experimental import pallas as pl
from jax.experimental.pallas import tpu as pltpu
from jax.experimental.pallas import tpu_sc as plsc

N_NODES_ = 10000
N_EDGES_ = 320000
NP = 10240
NCORES = 2
NSUB = 16
NTILES = NCORES * NSUB
EDGES_PER_TILE = 10240
EPAD = EDGES_PER_TILE * NTILES
BATCH = 128
NBATCH = EDGES_PER_TILE // BATCH
ROWS_PER_TILE = NP // NSUB

_MESH = plsc.VectorSubcoreMesh(core_axis_name="c", subcore_axis_name="s")


@functools.partial(
    pl.kernel,
    out_type=jax.ShapeDtypeStruct((NCORES, NP, 128), jnp.float32),
    mesh=_MESH,
    scratch_types=[
        pltpu.VMEM_SHARED((NP, 128), jnp.float32),
        pltpu.VMEM((NBATCH, BATCH), jnp.int32),
        pltpu.VMEM((BATCH, 128), jnp.float32),
    ],
)
def _deg_kernel(dst_hbm, zeros_hbm, ones_hbm, out_hbm, acc, dst_v, ones_v):
    c = lax.axis_index("c")
    s = lax.axis_index("s")
    wid = c * NSUB + s
    pltpu.sync_copy(zeros_hbm, acc.at[pl.ds(ROWS_PER_TILE * s, ROWS_PER_TILE)])
    pltpu.sync_copy(dst_hbm.at[pl.ds(wid * NBATCH, NBATCH)], dst_v)
    pltpu.sync_copy(ones_hbm, ones_v)
    plsc.subcore_barrier()

    def step(j, carry):
        pltpu.sync_copy(ones_v, acc.at[dst_v.at[j]], add=True)
        return carry

    lax.fori_loop(0, NBATCH, step, 0)
    plsc.subcore_barrier()
    sl = pl.ds(ROWS_PER_TILE * s, ROWS_PER_TILE)
    pltpu.sync_copy(acc.at[sl], out_hbm.at[c].at[sl])


def _make_agg(feat):
    @functools.partial(
        pl.kernel,
        out_type=jax.ShapeDtypeStruct((NCORES, NP, feat), jnp.float32),
        mesh=_MESH,
        scratch_types=[
            pltpu.VMEM_SHARED((NP, feat), jnp.float32),
            pltpu.VMEM((NBATCH, BATCH), jnp.int32),
            pltpu.VMEM((NBATCH, BATCH), jnp.int32),
            pltpu.VMEM((BATCH, feat), jnp.float32),
            pltpu.SemaphoreType.DMA,
        ],
    )
    def agg(x_hbm, src_hbm, dst_hbm, zeros_hbm, out_hbm,
            acc, src_v, dst_v, rows_v, sem):
        c = lax.axis_index("c")
        s = lax.axis_index("s")
        wid = c * NSUB + s
        pltpu.sync_copy(zeros_hbm,
                        acc.at[pl.ds(ROWS_PER_TILE * s, ROWS_PER_TILE)])
        pltpu.sync_copy(src_hbm.at[pl.ds(wid * NBATCH, NBATCH)], src_v)
        pltpu.sync_copy(dst_hbm.at[pl.ds(wid * NBATCH, NBATCH)], dst_v)
        plsc.subcore_barrier()

        def step(j, carry):
            pltpu.async_copy(x_hbm.at[src_v.at[j]], rows_v, sem).wait()
            pltpu.sync_copy(rows_v, acc.at[dst_v.at[j]], add=True)
            return carry

        lax.fori_loop(0, NBATCH, step, 0)
        plsc.subcore_barrier()
        sl = pl.ds(ROWS_PER_TILE * s, ROWS_PER_TILE)
        pltpu.sync_copy(acc.at[sl], out_hbm.at[c].at[sl])

    return agg


_agg128 = _make_agg(128)


BLK = 512
GRID = NP // BLK


def _scale_body(deg_ref, x_ref, dinv_ref, xt_ref):
    d = deg_ref[0, :, 0:1] + deg_ref[1, :, 0:1] + 1.0
    di = lax.rsqrt(d)
    dinv_ref[...] = di
    xt_ref[...] = x_ref[...] * di


def _scale_call(deg, x_pad):
    return pl.pallas_call(
        _scale_body,
        grid=(GRID,),
        in_specs=[
            pl.BlockSpec((NCORES, BLK, 128), lambda i: (0, i, 0)),
            pl.BlockSpec((BLK, 128), lambda i: (i, 0)),
        ],
        out_specs=[
            pl.BlockSpec((BLK, 1), lambda i: (i, 0)),
            pl.BlockSpec((BLK, 128), lambda i: (i, 0)),
        ],
        out_shape=[
            jax.ShapeDtypeStruct((NP, 1), jnp.float32),
            jax.ShapeDtypeStruct((NP, 128), jnp.float32),
        ],
    )(deg, x_pad)


def _fused_body(acc_ref, xt_ref, dinv_ref, w1_ref, b1_ref, w2_ref, out_ref):
    di = dinv_ref[...]
    z = (acc_ref[0] + acc_ref[1] + xt_ref[...]) * di
    h = jnp.dot(z, w1_ref[...], preferred_element_type=jnp.float32)
    h = jnp.maximum(h + b1_ref[...], 0.0)
    t = jnp.dot(h, w2_ref[...], preferred_element_type=jnp.float32)
    out_ref[...] = jnp.concatenate(
        [t * di, jnp.zeros((t.shape[0], 64), jnp.float32)], axis=1)


def _fused_call(acc1, xt, dinv, W1, b1, W2):
    return pl.pallas_call(
        _fused_body,
        grid=(GRID,),
        in_specs=[
            pl.BlockSpec((NCORES, BLK, 128), lambda i: (0, i, 0)),
            pl.BlockSpec((BLK, 128), lambda i: (i, 0)),
            pl.BlockSpec((BLK, 1), lambda i: (i, 0)),
            pl.BlockSpec((128, 1024), lambda i: (0, 0)),
            pl.BlockSpec((1, 1024), lambda i: (0, 0)),
            pl.BlockSpec((1024, 64), lambda i: (0, 0)),
        ],
        out_specs=pl.BlockSpec((BLK, 128), lambda i: (i, 0)),
        out_shape=jax.ShapeDtypeStruct((NP, 128), jnp.float32),
    )(acc1, xt, dinv, W1, b1.reshape(1, 1024), W2)


def _softmax_body(acc_ref, tt_ref, dinv_ref, b2_ref, out_ref):
    z128 = (acc_ref[0] + acc_ref[1] + tt_ref[...]) * dinv_ref[...]
    z = z128[:, :64] + b2_ref[...]
    m = jnp.max(z, axis=1, keepdims=True)
    e = jnp.exp(z - m)
    out_ref[...] = e / jnp.sum(e, axis=1, keepdims=True)


def _softmax_call(acc2, tt, dinv, b2):
    return pl.pallas_call(
        _softmax_body,
        grid=(GRID,),
        in_specs=[
            pl.BlockSpec((NCORES, BLK, 128), lambda i: (0, i, 0)),
            pl.BlockSpec((BLK, 128), lambda i: (i, 0)),
            pl.BlockSpec((BLK, 1), lambda i: (i, 0)),
            pl.BlockSpec((1, 64), lambda i: (0, 0)),
        ],
        out_specs=pl.BlockSpec((BLK, 64), lambda i: (i, 0)),
        out_shape=jax.ShapeDtypeStruct((NP, 64), jnp.float32),
    )(acc2, tt, dinv, b2.reshape(1, 64))


def kernel(x, edge_index, W1, b1, W2, b2):
    n = x.shape[0]
    e = edge_index.shape[1]
    src = edge_index[0].astype(jnp.int32)
    dst = edge_index[1].astype(jnp.int32)
    pad = EPAD - e
    src_p = jnp.concatenate([src, jnp.full((pad,), n, jnp.int32)])
    dst_p = jnp.concatenate([dst, jnp.full((pad,), n, jnp.int32)])
    src2d = src_p.reshape(EPAD // BATCH, BATCH)
    dst2d = dst_p.reshape(EPAD // BATCH, BATCH)
    x_pad = jnp.zeros((NP, 128), jnp.float32).at[:n].set(x)

    zeros128 = jnp.zeros((ROWS_PER_TILE, 128), jnp.float32)
    ones128 = jnp.ones((BATCH, 128), jnp.float32)

    deg = _deg_kernel(dst2d, zeros128, ones128)
    dinv, xt = _scale_call(deg, x_pad)
    acc1 = _agg128(xt, src2d, dst2d, zeros128)
    tt = _fused_call(acc1, xt, dinv, W1, b1, W2)
    acc2 = _agg128(tt, src2d, dst2d, zeros128)
    out = _softmax_call(acc2, tt, dinv, b2)
    return out[:n]


# speedup vs baseline: 10.6002x; 1.1057x over previous
import functools

import jax
import jax.numpy as jnp
from jax import lax
from jax.experimental import pallas as pl
from jax.experimental.pallas import tpu as pltpu
from jax.experimental.pallas import tpu_sc as plsc

N_NODES_ = 10000
N_EDGES_ = 320000
NP = 10240
NCORES = 2
NSUB = 16
NTILES = NCORES * NSUB
EDGES_PER_TILE = 10240
EPAD = EDGES_PER_TILE * NTILES
BATCH = 128
NBATCH = EDGES_PER_TILE // BATCH
ROWS_PER_TILE = NP // NSUB

_MESH = plsc.VectorSubcoreMesh(core_axis_name="c", subcore_axis_name="s")


@functools.partial(
    pl.kernel,
    out_type=jax.ShapeDtypeStruct((NCORES, NP, 128), jnp.float32),
    mesh=_MESH,
    scratch_types=[
        pltpu.VMEM_SHARED((NP, 128), jnp.float32),
        pltpu.VMEM((NBATCH, BATCH), jnp.int32),
        pltpu.VMEM((BATCH, 128), jnp.float32),
    ],
)
def _deg_kernel(dst_hbm, zeros_hbm, ones_hbm, out_hbm, acc, dst_v, ones_v):
    c = lax.axis_index("c")
    s = lax.axis_index("s")
    wid = c * NSUB + s
    pltpu.sync_copy(zeros_hbm, acc.at[pl.ds(ROWS_PER_TILE * s, ROWS_PER_TILE)])
    pltpu.sync_copy(dst_hbm.at[pl.ds(wid * NBATCH, NBATCH)], dst_v)
    pltpu.sync_copy(ones_hbm, ones_v)
    plsc.subcore_barrier()

    def step(j, carry):
        pltpu.sync_copy(ones_v, acc.at[dst_v.at[j]], add=True)
        return carry

    lax.fori_loop(0, NBATCH, step, 0)
    plsc.subcore_barrier()
    sl = pl.ds(ROWS_PER_TILE * s, ROWS_PER_TILE)
    pltpu.sync_copy(acc.at[sl], out_hbm.at[c].at[sl])


def _make_agg(feat):
    @functools.partial(
        pl.kernel,
        out_type=jax.ShapeDtypeStruct((NCORES, NP, feat), jnp.float32),
        mesh=_MESH,
        scratch_types=[
            pltpu.VMEM_SHARED((NP, feat), jnp.float32),
            pltpu.VMEM((NBATCH // 2, BATCH), jnp.int32),
            pltpu.VMEM((NBATCH // 2, BATCH), jnp.int32),
            pltpu.VMEM((BATCH, feat), jnp.float32),
            pltpu.VMEM((BATCH, feat), jnp.float32),
            pltpu.SemaphoreType.DMA,
            pltpu.SemaphoreType.DMA,
        ],
    )
    def agg(x_hbm, src_hbm, dst_hbm, zeros_hbm, out_hbm,
            acc, src_v, dst_v, rows_a, rows_b, sem_a, sem_b):
        c = lax.axis_index("c")
        s = lax.axis_index("s")
        wid = c * NSUB + s
        pltpu.sync_copy(zeros_hbm,
                        acc.at[pl.ds(ROWS_PER_TILE * s, ROWS_PER_TILE)])
        plsc.subcore_barrier()

        hb = NBATCH // 2
        for h in range(2):
            pltpu.sync_copy(
                src_hbm.at[pl.ds(wid * NBATCH + h * hb, hb)], src_v)
            pltpu.sync_copy(
                dst_hbm.at[pl.ds(wid * NBATCH + h * hb, hb)], dst_v)
            pltpu.async_copy(x_hbm.at[src_v.at[0]], rows_a, sem_a)

            def step(i, carry):
                j0 = 2 * i
                pltpu.async_copy(x_hbm.at[src_v.at[j0 + 1]], rows_b, sem_b)
                pltpu.make_async_copy(
                    x_hbm.at[src_v.at[j0]], rows_a, sem_a).wait()
                pltpu.sync_copy(rows_a, acc.at[dst_v.at[j0]], add=True)

                @pl.when(i < hb // 2 - 1)
                def _():
                    pltpu.async_copy(
                        x_hbm.at[src_v.at[j0 + 2]], rows_a, sem_a)

                pltpu.make_async_copy(
                    x_hbm.at[src_v.at[j0 + 1]], rows_b, sem_b).wait()
                pltpu.sync_copy(rows_b, acc.at[dst_v.at[j0 + 1]], add=True)
                return carry

            lax.fori_loop(0, hb // 2, step, 0)
        plsc.subcore_barrier()
        sl = pl.ds(ROWS_PER_TILE * s, ROWS_PER_TILE)
        pltpu.sync_copy(acc.at[sl], out_hbm.at[c].at[sl])

    return agg


_agg128 = _make_agg(128)


BLK = 512
GRID = NP // BLK


def _scale_body(deg_ref, x_ref, dinv_ref, xt_ref):
    d = deg_ref[0, :, 0:1] + deg_ref[1, :, 0:1] + 1.0
    di = lax.rsqrt(d)
    dinv_ref[...] = di
    xt_ref[...] = x_ref[...] * di


def _scale_call(deg, x_pad):
    return pl.pallas_call(
        _scale_body,
        grid=(GRID,),
        in_specs=[
            pl.BlockSpec((NCORES, BLK, 128), lambda i: (0, i, 0)),
            pl.BlockSpec((BLK, 128), lambda i: (i, 0)),
        ],
        out_specs=[
            pl.BlockSpec((BLK, 1), lambda i: (i, 0)),
            pl.BlockSpec((BLK, 128), lambda i: (i, 0)),
        ],
        out_shape=[
            jax.ShapeDtypeStruct((NP, 1), jnp.float32),
            jax.ShapeDtypeStruct((NP, 128), jnp.float32),
        ],
    )(deg, x_pad)


def _fused_body(acc_ref, xt_ref, dinv_ref, w1_ref, b1_ref, w2_ref, out_ref):
    di = dinv_ref[...]
    z = (acc_ref[0] + acc_ref[1] + xt_ref[...]) * di
    h = jnp.dot(z, w1_ref[...], preferred_element_type=jnp.float32)
    h = jnp.maximum(h + b1_ref[...], 0.0)
    t = jnp.dot(h, w2_ref[...], preferred_element_type=jnp.float32)
    out_ref[...] = jnp.concatenate(
        [t * di, jnp.zeros((t.shape[0], 64), jnp.float32)], axis=1)


def _fused_call(acc1, xt, dinv, W1, b1, W2):
    return pl.pallas_call(
        _fused_body,
        grid=(GRID,),
        in_specs=[
            pl.BlockSpec((NCORES, BLK, 128), lambda i: (0, i, 0)),
            pl.BlockSpec((BLK, 128), lambda i: (i, 0)),
            pl.BlockSpec((BLK, 1), lambda i: (i, 0)),
            pl.BlockSpec((128, 1024), lambda i: (0, 0)),
            pl.BlockSpec((1, 1024), lambda i: (0, 0)),
            pl.BlockSpec((1024, 64), lambda i: (0, 0)),
        ],
        out_specs=pl.BlockSpec((BLK, 128), lambda i: (i, 0)),
        out_shape=jax.ShapeDtypeStruct((NP, 128), jnp.float32),
    )(acc1, xt, dinv, W1, b1.reshape(1, 1024), W2)


def _softmax_body(acc_ref, tt_ref, dinv_ref, b2_ref, out_ref):
    z128 = (acc_ref[0] + acc_ref[1] + tt_ref[...]) * dinv_ref[...]
    z = z128[:, :64] + b2_ref[...]
    m = jnp.max(z, axis=1, keepdims=True)
    e = jnp.exp(z - m)
    out_ref[...] = e / jnp.sum(e, axis=1, keepdims=True)


def _softmax_call(acc2, tt, dinv, b2):
    return pl.pallas_call(
        _softmax_body,
        grid=(GRID,),
        in_specs=[
            pl.BlockSpec((NCORES, BLK, 128), lambda i: (0, i, 0)),
            pl.BlockSpec((BLK, 128), lambda i: (i, 0)),
            pl.BlockSpec((BLK, 1), lambda i: (i, 0)),
            pl.BlockSpec((1, 64), lambda i: (0, 0)),
        ],
        out_specs=pl.BlockSpec((BLK, 64), lambda i: (i, 0)),
        out_shape=jax.ShapeDtypeStruct((NP, 64), jnp.float32),
    )(acc2, tt, dinv, b2.reshape(1, 64))


def kernel(x, edge_index, W1, b1, W2, b2):
    n = x.shape[0]
    e = edge_index.shape[1]
    src = edge_index[0].astype(jnp.int32)
    dst = edge_index[1].astype(jnp.int32)
    pad = EPAD - e
    src_p = jnp.concatenate([src, jnp.full((pad,), n, jnp.int32)])
    dst_p = jnp.concatenate([dst, jnp.full((pad,), n, jnp.int32)])
    src2d = src_p.reshape(EPAD // BATCH, BATCH)
    dst2d = dst_p.reshape(EPAD // BATCH, BATCH)
    x_pad = jnp.zeros((NP, 128), jnp.float32).at[:n].set(x)

    zeros128 = jnp.zeros((ROWS_PER_TILE, 128), jnp.float32)
    ones128 = jnp.ones((BATCH, 128), jnp.float32)

    deg = _deg_kernel(dst2d, zeros128, ones128)
    dinv, xt = _scale_call(deg, x_pad)
    acc1 = _agg128(xt, src2d, dst2d, zeros128)
    tt = _fused_call(acc1, xt, dinv, W1, b1, W2)
    acc2 = _agg128(tt, src2d, dst2d, zeros128)
    out = _softmax_call(acc2, tt, dinv, b2)
    return out[:n]


# speedup vs baseline: 10.9297x; 1.0311x over previous
import functools

import jax
import jax.numpy as jnp
from jax import lax
from jax.experimental import pallas as pl
from jax.experimental.pallas import tpu as pltpu
from jax.experimental.pallas import tpu_sc as plsc

N_NODES_ = 10000
N_EDGES_ = 320000
NP = 10240
NCORES = 2
NSUB = 16
NTILES = NCORES * NSUB
EDGES_PER_TILE = 10240
EPAD = EDGES_PER_TILE * NTILES
BATCH = 128
NBATCH = EDGES_PER_TILE // BATCH
ROWS_PER_TILE = NP // NSUB

_MESH = plsc.VectorSubcoreMesh(core_axis_name="c", subcore_axis_name="s")


@functools.partial(
    pl.kernel,
    out_type=jax.ShapeDtypeStruct((NCORES, NP, 128), jnp.float32),
    mesh=_MESH,
    scratch_types=[
        pltpu.VMEM_SHARED((NP, 128), jnp.float32),
        pltpu.VMEM((NBATCH, BATCH), jnp.int32),
        pltpu.VMEM((BATCH, 128), jnp.float32),
    ],
)
def _deg_kernel(dst_hbm, zeros_hbm, ones_hbm, out_hbm, acc, dst_v, ones_v):
    c = lax.axis_index("c")
    s = lax.axis_index("s")
    wid = c * NSUB + s
    pltpu.sync_copy(zeros_hbm, acc.at[pl.ds(ROWS_PER_TILE * s, ROWS_PER_TILE)])
    pltpu.sync_copy(dst_hbm.at[pl.ds(wid * NBATCH, NBATCH)], dst_v)
    pltpu.sync_copy(ones_hbm, ones_v)
    plsc.subcore_barrier()

    def step(j, carry):
        pltpu.sync_copy(ones_v, acc.at[dst_v.at[j]], add=True)
        return carry

    lax.fori_loop(0, NBATCH, step, 0)
    plsc.subcore_barrier()
    sl = pl.ds(ROWS_PER_TILE * s, ROWS_PER_TILE)
    pltpu.sync_copy(acc.at[sl], out_hbm.at[c].at[sl])


SBATCH = 64
NB64 = EDGES_PER_TILE // SBATCH
NBUF = 4


def _make_agg(feat):
    @functools.partial(
        pl.kernel,
        out_type=jax.ShapeDtypeStruct((NCORES, NP, feat), jnp.float32),
        mesh=_MESH,
        scratch_types=[
            pltpu.VMEM_SHARED((NP, feat), jnp.float32),
            pltpu.VMEM((NB64 // 4, SBATCH), jnp.int32),
            pltpu.VMEM((NB64 // 4, SBATCH), jnp.int32),
            [pltpu.VMEM((SBATCH, feat), jnp.float32) for _ in range(NBUF)],
            [pltpu.SemaphoreType.DMA for _ in range(NBUF)],
        ],
    )
    def agg(x_hbm, src_hbm, dst_hbm, zeros_hbm, out_hbm,
            acc, src_v, dst_v, bufs, sems):
        c = lax.axis_index("c")
        s = lax.axis_index("s")
        wid = c * NSUB + s
        pltpu.sync_copy(zeros_hbm,
                        acc.at[pl.ds(ROWS_PER_TILE * s, ROWS_PER_TILE)])
        plsc.subcore_barrier()

        hb = NB64 // 4
        for h in range(4):
            pltpu.sync_copy(
                src_hbm.at[pl.ds(wid * NB64 + h * hb, hb)], src_v)
            pltpu.sync_copy(
                dst_hbm.at[pl.ds(wid * NB64 + h * hb, hb)], dst_v)
            for q in range(NBUF - 1):
                pltpu.async_copy(x_hbm.at[src_v.at[q]], bufs[q], sems[q])

            def step(i, carry):
                for q in range(NBUF):
                    j = NBUF * i + q
                    pltpu.make_async_copy(
                        x_hbm.at[src_v.at[j]], bufs[q], sems[q]).wait()
                    pltpu.sync_copy(bufs[q], acc.at[dst_v.at[j]], add=True)
                    qn = (q + NBUF - 1) % NBUF

                    @pl.when(j + NBUF - 1 < hb)
                    def _():
                        pltpu.async_copy(
                            x_hbm.at[src_v.at[j + NBUF - 1]],
                            bufs[qn], sems[qn])
                return carry

            lax.fori_loop(0, hb // NBUF, step, 0)
        plsc.subcore_barrier()
        sl = pl.ds(ROWS_PER_TILE * s, ROWS_PER_TILE)
        pltpu.sync_copy(acc.at[sl], out_hbm.at[c].at[sl])

    return agg


_agg128 = _make_agg(128)


BLK = 512
GRID = NP // BLK


def _scale_body(deg_ref, x_ref, dinv_ref, xt_ref):
    d = deg_ref[0, :, 0:1] + deg_ref[1, :, 0:1] + 1.0
    di = lax.rsqrt(d)
    dinv_ref[...] = di
    xt_ref[...] = x_ref[...] * di


def _scale_call(deg, x_pad):
    return pl.pallas_call(
        _scale_body,
        grid=(GRID,),
        in_specs=[
            pl.BlockSpec((NCORES, BLK, 128), lambda i: (0, i, 0)),
            pl.BlockSpec((BLK, 128), lambda i: (i, 0)),
        ],
        out_specs=[
            pl.BlockSpec((BLK, 1), lambda i: (i, 0)),
            pl.BlockSpec((BLK, 128), lambda i: (i, 0)),
        ],
        out_shape=[
            jax.ShapeDtypeStruct((NP, 1), jnp.float32),
            jax.ShapeDtypeStruct((NP, 128), jnp.float32),
        ],
    )(deg, x_pad)


def _fused_body(acc_ref, xt_ref, dinv_ref, w1_ref, b1_ref, w2_ref, out_ref):
    di = dinv_ref[...]
    z = (acc_ref[0] + acc_ref[1] + xt_ref[...]) * di
    h = jnp.dot(z, w1_ref[...], preferred_element_type=jnp.float32)
    h = jnp.maximum(h + b1_ref[...], 0.0)
    t = jnp.dot(h, w2_ref[...], preferred_element_type=jnp.float32)
    out_ref[...] = jnp.concatenate(
        [t * di, jnp.zeros((t.shape[0], 64), jnp.float32)], axis=1)


def _fused_call(acc1, xt, dinv, W1, b1, W2):
    return pl.pallas_call(
        _fused_body,
        grid=(GRID,),
        in_specs=[
            pl.BlockSpec((NCORES, BLK, 128), lambda i: (0, i, 0)),
            pl.BlockSpec((BLK, 128), lambda i: (i, 0)),
            pl.BlockSpec((BLK, 1), lambda i: (i, 0)),
            pl.BlockSpec((128, 1024), lambda i: (0, 0)),
            pl.BlockSpec((1, 1024), lambda i: (0, 0)),
            pl.BlockSpec((1024, 64), lambda i: (0, 0)),
        ],
        out_specs=pl.BlockSpec((BLK, 128), lambda i: (i, 0)),
        out_shape=jax.ShapeDtypeStruct((NP, 128), jnp.float32),
    )(acc1, xt, dinv, W1, b1.reshape(1, 1024), W2)


def _softmax_body(acc_ref, tt_ref, dinv_ref, b2_ref, out_ref):
    z128 = (acc_ref[0] + acc_ref[1] + tt_ref[...]) * dinv_ref[...]
    z = z128[:, :64] + b2_ref[...]
    m = jnp.max(z, axis=1, keepdims=True)
    e = jnp.exp(z - m)
    out_ref[...] = e / jnp.sum(e, axis=1, keepdims=True)


def _softmax_call(acc2, tt, dinv, b2):
    return pl.pallas_call(
        _softmax_body,
        grid=(GRID,),
        in_specs=[
            pl.BlockSpec((NCORES, BLK, 128), lambda i: (0, i, 0)),
            pl.BlockSpec((BLK, 128), lambda i: (i, 0)),
            pl.BlockSpec((BLK, 1), lambda i: (i, 0)),
            pl.BlockSpec((1, 64), lambda i: (0, 0)),
        ],
        out_specs=pl.BlockSpec((BLK, 64), lambda i: (i, 0)),
        out_shape=jax.ShapeDtypeStruct((NP, 64), jnp.float32),
    )(acc2, tt, dinv, b2.reshape(1, 64))


def kernel(x, edge_index, W1, b1, W2, b2):
    n = x.shape[0]
    e = edge_index.shape[1]
    src = edge_index[0].astype(jnp.int32)
    dst = edge_index[1].astype(jnp.int32)
    pad = EPAD - e
    src_p = jnp.concatenate([src, jnp.full((pad,), n, jnp.int32)])
    dst_p = jnp.concatenate([dst, jnp.full((pad,), n, jnp.int32)])
    src2d = src_p.reshape(EPAD // BATCH, BATCH)
    dst2d = dst_p.reshape(EPAD // BATCH, BATCH)
    src64 = src_p.reshape(EPAD // SBATCH, SBATCH)
    dst64 = dst_p.reshape(EPAD // SBATCH, SBATCH)
    x_pad = jnp.zeros((NP, 128), jnp.float32).at[:n].set(x)

    zeros128 = jnp.zeros((ROWS_PER_TILE, 128), jnp.float32)
    ones128 = jnp.ones((BATCH, 128), jnp.float32)

    deg = _deg_kernel(dst2d, zeros128, ones128)
    dinv, xt = _scale_call(deg, x_pad)
    acc1 = _agg128(xt, src64, dst64, zeros128)
    tt = _fused_call(acc1, xt, dinv, W1, b1, W2)
    acc2 = _agg128(tt, src64, dst64, zeros128)
    out = _softmax_call(acc2, tt, dinv, b2)
    return out[:n]


# speedup vs baseline: 11.7869x; 1.0784x over previous
import functools

import jax
import jax.numpy as jnp
from jax import lax
from jax.experimental import pallas as pl
from jax.experimental.pallas import tpu as pltpu
from jax.experimental.pallas import tpu_sc as plsc

N_NODES_ = 10000
N_EDGES_ = 320000
NP = 10240
NCORES = 2
NSUB = 16
NTILES = NCORES * NSUB
EDGES_PER_TILE = 10240
EPAD = EDGES_PER_TILE * NTILES
BATCH = 128
NBATCH = EDGES_PER_TILE // BATCH
ROWS_PER_TILE = NP // NSUB

_MESH = plsc.VectorSubcoreMesh(core_axis_name="c", subcore_axis_name="s")


@functools.partial(
    pl.kernel,
    out_type=jax.ShapeDtypeStruct((NCORES, NP, 128), jnp.float32),
    mesh=_MESH,
    scratch_types=[
        pltpu.VMEM_SHARED((NP, 128), jnp.float32),
        pltpu.VMEM((NBATCH, BATCH), jnp.int32),
        pltpu.VMEM((BATCH, 128), jnp.float32),
    ],
)
def _deg_kernel(dst_hbm, zeros_hbm, ones_hbm, out_hbm, acc, dst_v, ones_v):
    c = lax.axis_index("c")
    s = lax.axis_index("s")
    wid = c * NSUB + s
    pltpu.sync_copy(zeros_hbm, acc.at[pl.ds(ROWS_PER_TILE * s, ROWS_PER_TILE)])
    pltpu.sync_copy(dst_hbm.at[pl.ds(wid * NBATCH, NBATCH)], dst_v)
    pltpu.sync_copy(ones_hbm, ones_v)
    plsc.subcore_barrier()

    def step(j, carry):
        pltpu.sync_copy(ones_v, acc.at[dst_v.at[j]], add=True)
        return carry

    lax.fori_loop(0, NBATCH, step, 0)
    plsc.subcore_barrier()
    sl = pl.ds(ROWS_PER_TILE * s, ROWS_PER_TILE)
    pltpu.sync_copy(acc.at[sl], out_hbm.at[c].at[sl])


SBATCH = 64
NBTOT = EPAD // SBATCH
NST0, NST1 = 8, 2
ST = 32
NBUF = 4
assert (NST0 + NST1) * ST * NSUB == NBTOT


def _make_agg(feat):
    @functools.partial(
        pl.kernel,
        out_type=jax.ShapeDtypeStruct((NCORES, NP, feat), jnp.float32),
        mesh=_MESH,
        scratch_types=[
            pltpu.VMEM_SHARED((NP, feat), jnp.float32),
            pltpu.VMEM((ST, SBATCH), jnp.int32),
            pltpu.VMEM((ST, SBATCH), jnp.int32),
            [pltpu.VMEM((SBATCH, feat), jnp.float32) for _ in range(NBUF)],
            [pltpu.SemaphoreType.DMA for _ in range(NBUF)],
        ],
    )
    def agg(x_hbm, src_hbm, dst_hbm, zeros_hbm, out_hbm,
            acc, src_v, dst_v, bufs, sems):
        c = lax.axis_index("c")
        s = lax.axis_index("s")
        pltpu.sync_copy(zeros_hbm,
                        acc.at[pl.ds(ROWS_PER_TILE * s, ROWS_PER_TILE)])
        plsc.subcore_barrier()

        nst = jnp.where(c == 0, NST0, NST1)
        row0 = jnp.where(c == 0, ST * NST0 * s,
                         ST * NST0 * NSUB + ST * NST1 * s)

        def stage(h, carry):
            base = row0 + h * ST
            pltpu.sync_copy(src_hbm.at[pl.ds(base, ST)], src_v)
            pltpu.sync_copy(dst_hbm.at[pl.ds(base, ST)], dst_v)
            for q in range(NBUF - 1):
                pltpu.async_copy(x_hbm.at[src_v.at[q]], bufs[q], sems[q])

            def step(i, carry2):
                for q in range(NBUF):
                    j = NBUF * i + q
                    pltpu.make_async_copy(
                        x_hbm.at[src_v.at[j]], bufs[q], sems[q]).wait()
                    pltpu.sync_copy(bufs[q], acc.at[dst_v.at[j]], add=True)
                    qn = (q + NBUF - 1) % NBUF

                    @pl.when(j + NBUF - 1 < ST)
                    def _():
                        pltpu.async_copy(
                            x_hbm.at[src_v.at[j + NBUF - 1]],
                            bufs[qn], sems[qn])
                return carry2

            lax.fori_loop(0, ST // NBUF, step, 0)
            return carry

        lax.fori_loop(0, nst, stage, 0)
        plsc.subcore_barrier()
        sl = pl.ds(ROWS_PER_TILE * s, ROWS_PER_TILE)
        pltpu.sync_copy(acc.at[sl], out_hbm.at[c].at[sl])

    return agg


_agg128 = _make_agg(128)


BLK = 512
GRID = NP // BLK


def _scale_body(deg_ref, x_ref, dinv_ref, xt_ref):
    d = deg_ref[0, :, 0:1] + deg_ref[1, :, 0:1] + 1.0
    di = lax.rsqrt(d)
    dinv_ref[...] = di
    xt_ref[...] = x_ref[...] * di


def _scale_call(deg, x_pad):
    return pl.pallas_call(
        _scale_body,
        grid=(GRID,),
        in_specs=[
            pl.BlockSpec((NCORES, BLK, 128), lambda i: (0, i, 0)),
            pl.BlockSpec((BLK, 128), lambda i: (i, 0)),
        ],
        out_specs=[
            pl.BlockSpec((BLK, 1), lambda i: (i, 0)),
            pl.BlockSpec((BLK, 128), lambda i: (i, 0)),
        ],
        out_shape=[
            jax.ShapeDtypeStruct((NP, 1), jnp.float32),
            jax.ShapeDtypeStruct((NP, 128), jnp.float32),
        ],
    )(deg, x_pad)


def _fused_body(acc_ref, xt_ref, dinv_ref, w1_ref, b1_ref, w2_ref, out_ref):
    di = dinv_ref[...]
    z = (acc_ref[0] + acc_ref[1] + xt_ref[...]) * di
    h = jnp.dot(z, w1_ref[...], preferred_element_type=jnp.float32)
    h = jnp.maximum(h + b1_ref[...], 0.0)
    t = jnp.dot(h, w2_ref[...], preferred_element_type=jnp.float32)
    out_ref[...] = jnp.concatenate(
        [t * di, jnp.zeros((t.shape[0], 64), jnp.float32)], axis=1)


def _fused_call(acc1, xt, dinv, W1, b1, W2):
    return pl.pallas_call(
        _fused_body,
        grid=(GRID,),
        in_specs=[
            pl.BlockSpec((NCORES, BLK, 128), lambda i: (0, i, 0)),
            pl.BlockSpec((BLK, 128), lambda i: (i, 0)),
            pl.BlockSpec((BLK, 1), lambda i: (i, 0)),
            pl.BlockSpec((128, 1024), lambda i: (0, 0)),
            pl.BlockSpec((1, 1024), lambda i: (0, 0)),
            pl.BlockSpec((1024, 64), lambda i: (0, 0)),
        ],
        out_specs=pl.BlockSpec((BLK, 128), lambda i: (i, 0)),
        out_shape=jax.ShapeDtypeStruct((NP, 128), jnp.float32),
    )(acc1, xt, dinv, W1, b1.reshape(1, 1024), W2)


def _softmax_body(acc_ref, tt_ref, dinv_ref, b2_ref, out_ref):
    z128 = (acc_ref[0] + acc_ref[1] + tt_ref[...]) * dinv_ref[...]
    z = z128[:, :64] + b2_ref[...]
    m = jnp.max(z, axis=1, keepdims=True)
    e = jnp.exp(z - m)
    out_ref[...] = e / jnp.sum(e, axis=1, keepdims=True)


def _softmax_call(acc2, tt, dinv, b2):
    return pl.pallas_call(
        _softmax_body,
        grid=(GRID,),
        in_specs=[
            pl.BlockSpec((NCORES, BLK, 128), lambda i: (0, i, 0)),
            pl.BlockSpec((BLK, 128), lambda i: (i, 0)),
            pl.BlockSpec((BLK, 1), lambda i: (i, 0)),
            pl.BlockSpec((1, 64), lambda i: (0, 0)),
        ],
        out_specs=pl.BlockSpec((BLK, 64), lambda i: (i, 0)),
        out_shape=jax.ShapeDtypeStruct((NP, 64), jnp.float32),
    )(acc2, tt, dinv, b2.reshape(1, 64))


def kernel(x, edge_index, W1, b1, W2, b2):
    n = x.shape[0]
    e = edge_index.shape[1]
    src = edge_index[0].astype(jnp.int32)
    dst = edge_index[1].astype(jnp.int32)
    pad = EPAD - e
    src_p = jnp.concatenate([src, jnp.full((pad,), n, jnp.int32)])
    dst_p = jnp.concatenate([dst, jnp.full((pad,), n, jnp.int32)])
    src2d = src_p.reshape(EPAD // BATCH, BATCH)
    dst2d = dst_p.reshape(EPAD // BATCH, BATCH)
    src64 = src_p.reshape(EPAD // SBATCH, SBATCH)
    dst64 = dst_p.reshape(EPAD // SBATCH, SBATCH)
    x_pad = jnp.zeros((NP, 128), jnp.float32).at[:n].set(x)

    zeros128 = jnp.zeros((ROWS_PER_TILE, 128), jnp.float32)
    ones128 = jnp.ones((BATCH, 128), jnp.float32)

    deg = _deg_kernel(dst2d, zeros128, ones128)
    dinv, xt = _scale_call(deg, x_pad)
    acc1 = _agg128(xt, src64, dst64, zeros128)
    tt = _fused_call(acc1, xt, dinv, W1, b1, W2)
    acc2 = _agg128(tt, src64, dst64, zeros128)
    out = _softmax_call(acc2, tt, dinv, b2)
    return out[:n]


# speedup vs baseline: 34.9970x; 2.9691x over previous
import functools

import jax
import jax.numpy as jnp
from jax import lax
from jax.experimental import pallas as pl
from jax.experimental.pallas import tpu as pltpu
from jax.experimental.pallas import tpu_sc as plsc

N_NODES_ = 10000
N_EDGES_ = 320000
NP = 10240
NCORES = 2
NSUB = 16
NTILES = NCORES * NSUB
EDGES_PER_TILE = 10240
EPAD = EDGES_PER_TILE * NTILES
BATCH = 128
NBATCH = EDGES_PER_TILE // BATCH
ROWS_PER_TILE = NP // NSUB

_MESH = plsc.VectorSubcoreMesh(core_axis_name="c", subcore_axis_name="s")


@functools.partial(
    pl.kernel,
    out_type=[jax.ShapeDtypeStruct((NP,), jnp.float32),
              jax.ShapeDtypeStruct((NP,), jnp.float32)],
    mesh=_MESH,
    scratch_types=[
        pltpu.VMEM_SHARED((NSUB, NP), jnp.float32),
        pltpu.VMEM((NBATCH, BATCH), jnp.int32),
        pltpu.VMEM((NP,), jnp.float32),
        pltpu.VMEM((ROWS_PER_TILE,), jnp.float32),
        pltpu.VMEM((ROWS_PER_TILE,), jnp.float32),
    ],
    compiler_params=pltpu.CompilerParams(needs_layout_passes=False),
)
def _deg_kernel(dst_hbm, out0, out1, sh, dst_v, hist, red_a, red_b):
    c = lax.axis_index("c")
    s = lax.axis_index("s")
    wid = c * NSUB + s
    pltpu.sync_copy(dst_hbm.at[pl.ds(wid * NBATCH, NBATCH)], dst_v)

    zero16 = jnp.zeros((16,), jnp.float32)
    one16 = jnp.ones((16,), jnp.float32)

    def zstep(i, carry):
        hist[pl.ds(i * 16, 16)] = zero16
        return carry

    lax.fori_loop(0, NP // 16, zstep, 0)

    def hstep(j, carry):
        for k in range(BATCH // 16):
            idx = dst_v[j, pl.ds(k * 16, 16)]
            plsc.addupdate_scatter(hist, [idx], one16)
        return carry

    lax.fori_loop(0, NBATCH, hstep, 0)

    pltpu.sync_copy(hist, sh.at[s])
    plsc.subcore_barrier()

    sl = pl.ds(ROWS_PER_TILE * s, ROWS_PER_TILE)
    pltpu.sync_copy(sh.at[0].at[sl], red_a)
    for k in range(1, NSUB):
        pltpu.sync_copy(sh.at[k].at[sl], red_b)

        def astep(m, carry):
            red_a[pl.ds(m * 16, 16)] = (red_a[pl.ds(m * 16, 16)]
                                        + red_b[pl.ds(m * 16, 16)])
            return carry

        lax.fori_loop(0, ROWS_PER_TILE // 16, astep, 0)

    @pl.when(c == 0)
    def _():
        pltpu.sync_copy(red_a, out0.at[sl])

    @pl.when(c == 1)
    def _():
        pltpu.sync_copy(red_a, out1.at[sl])


SBATCH = 64
NBTOT = EPAD // SBATCH
NST0, NST1 = 5, 5
ST = 32
NBUF = 4
assert (NST0 + NST1) * ST * NSUB == NBTOT


def _make_agg(feat):
    @functools.partial(
        pl.kernel,
        out_type=jax.ShapeDtypeStruct((NCORES, NP, feat), jnp.float32),
        mesh=_MESH,
        scratch_types=[
            pltpu.VMEM_SHARED((NP, feat), jnp.float32),
            pltpu.VMEM((ST, SBATCH), jnp.int32),
            pltpu.VMEM((ST, SBATCH), jnp.int32),
            [pltpu.VMEM((SBATCH, feat), jnp.float32) for _ in range(NBUF)],
            [pltpu.SemaphoreType.DMA for _ in range(NBUF)],
        ],
    )
    def agg(x_hbm, src_hbm, dst_hbm, zeros_hbm, out_hbm,
            acc, src_v, dst_v, bufs, sems):
        c = lax.axis_index("c")
        s = lax.axis_index("s")
        pltpu.sync_copy(zeros_hbm,
                        acc.at[pl.ds(ROWS_PER_TILE * s, ROWS_PER_TILE)])
        plsc.subcore_barrier()

        nst = jnp.where(c == 0, NST0, NST1)
        row0 = jnp.where(c == 0, ST * NST0 * s,
                         ST * NST0 * NSUB + ST * NST1 * s)

        def stage(h, carry):
            base = row0 + h * ST
            pltpu.sync_copy(src_hbm.at[pl.ds(base, ST)], src_v)
            pltpu.sync_copy(dst_hbm.at[pl.ds(base, ST)], dst_v)
            for q in range(NBUF - 1):
                pltpu.async_copy(x_hbm.at[src_v.at[q]], bufs[q], sems[q])

            def step(i, carry2):
                for q in range(NBUF):
                    j = NBUF * i + q
                    pltpu.make_async_copy(
                        x_hbm.at[src_v.at[j]], bufs[q], sems[q]).wait()
                    pltpu.sync_copy(bufs[q], acc.at[dst_v.at[j]], add=True)
                    qn = (q + NBUF - 1) % NBUF

                    @pl.when(j + NBUF - 1 < ST)
                    def _():
                        pltpu.async_copy(
                            x_hbm.at[src_v.at[j + NBUF - 1]],
                            bufs[qn], sems[qn])
                return carry2

            lax.fori_loop(0, ST // NBUF, step, 0)
            return carry

        lax.fori_loop(0, nst, stage, 0)
        plsc.subcore_barrier()
        sl = pl.ds(ROWS_PER_TILE * s, ROWS_PER_TILE)
        pltpu.sync_copy(acc.at[sl], out_hbm.at[c].at[sl])

    return agg


_agg128 = _make_agg(128)


BLK = 512
GRID = NP // BLK


def _scale_body(deg0_ref, deg1_ref, x_ref, dinv_ref, xt_ref):
    d = deg0_ref[...] + deg1_ref[...] + 1.0
    di = lax.rsqrt(d)
    dinv_ref[...] = di
    xt_ref[...] = x_ref[...] * di


def _scale_call(deg0, deg1, x_pad):
    return pl.pallas_call(
        _scale_body,
        grid=(GRID,),
        in_specs=[
            pl.BlockSpec((BLK, 1), lambda i: (i, 0)),
            pl.BlockSpec((BLK, 1), lambda i: (i, 0)),
            pl.BlockSpec((BLK, 128), lambda i: (i, 0)),
        ],
        out_specs=[
            pl.BlockSpec((BLK, 1), lambda i: (i, 0)),
            pl.BlockSpec((BLK, 128), lambda i: (i, 0)),
        ],
        out_shape=[
            jax.ShapeDtypeStruct((NP, 1), jnp.float32),
            jax.ShapeDtypeStruct((NP, 128), jnp.float32),
        ],
    )(deg0, deg1, x_pad)


def _fused_body(acc_ref, xt_ref, dinv_ref, w1_ref, b1_ref, w2_ref, out_ref):
    di = dinv_ref[...]
    z = (acc_ref[0] + acc_ref[1] + xt_ref[...]) * di
    h = jnp.dot(z, w1_ref[...], preferred_element_type=jnp.float32)
    h = jnp.maximum(h + b1_ref[...], 0.0)
    t = jnp.dot(h, w2_ref[...], preferred_element_type=jnp.float32)
    out_ref[...] = jnp.concatenate(
        [t * di, jnp.zeros((t.shape[0], 64), jnp.float32)], axis=1)


def _fused_call(acc1, xt, dinv, W1, b1, W2):
    return pl.pallas_call(
        _fused_body,
        grid=(GRID,),
        in_specs=[
            pl.BlockSpec((NCORES, BLK, 128), lambda i: (0, i, 0)),
            pl.BlockSpec((BLK, 128), lambda i: (i, 0)),
            pl.BlockSpec((BLK, 1), lambda i: (i, 0)),
            pl.BlockSpec((128, 1024), lambda i: (0, 0)),
            pl.BlockSpec((1, 1024), lambda i: (0, 0)),
            pl.BlockSpec((1024, 64), lambda i: (0, 0)),
        ],
        out_specs=pl.BlockSpec((BLK, 128), lambda i: (i, 0)),
        out_shape=jax.ShapeDtypeStruct((NP, 128), jnp.float32),
    )(acc1, xt, dinv, W1, b1.reshape(1, 1024), W2)


def _softmax_body(acc_ref, tt_ref, dinv_ref, b2_ref, out_ref):
    z128 = (acc_ref[0] + acc_ref[1] + tt_ref[...]) * dinv_ref[...]
    z = z128[:, :64] + b2_ref[...]
    m = jnp.max(z, axis=1, keepdims=True)
    e = jnp.exp(z - m)
    out_ref[...] = e / jnp.sum(e, axis=1, keepdims=True)


def _softmax_call(acc2, tt, dinv, b2):
    return pl.pallas_call(
        _softmax_body,
        grid=(GRID,),
        in_specs=[
            pl.BlockSpec((NCORES, BLK, 128), lambda i: (0, i, 0)),
            pl.BlockSpec((BLK, 128), lambda i: (i, 0)),
            pl.BlockSpec((BLK, 1), lambda i: (i, 0)),
            pl.BlockSpec((1, 64), lambda i: (0, 0)),
        ],
        out_specs=pl.BlockSpec((BLK, 64), lambda i: (i, 0)),
        out_shape=jax.ShapeDtypeStruct((NP, 64), jnp.float32),
    )(acc2, tt, dinv, b2.reshape(1, 64))


def kernel(x, edge_index, W1, b1, W2, b2):
    n = x.shape[0]
    e = edge_index.shape[1]
    src = edge_index[0].astype(jnp.int32)
    dst = edge_index[1].astype(jnp.int32)
    pad = EPAD - e
    pad_idx = n + (jnp.arange(pad, dtype=jnp.int32) % (NP - n))
    src_p = jnp.concatenate([src, pad_idx])
    dst_p = jnp.concatenate([dst, pad_idx])
    src2d = src_p.reshape(EPAD // BATCH, BATCH)
    dst2d = dst_p.reshape(EPAD // BATCH, BATCH)
    src64 = src_p.reshape(EPAD // SBATCH, SBATCH)
    dst64 = dst_p.reshape(EPAD // SBATCH, SBATCH)
    x_pad = jnp.zeros((NP, 128), jnp.float32).at[:n].set(x)

    zeros128 = jnp.zeros((ROWS_PER_TILE, 128), jnp.float32)

    deg0, deg1 = _deg_kernel(dst2d)
    dinv, xt = _scale_call(deg0.reshape(NP, 1), deg1.reshape(NP, 1), x_pad)
    acc1 = _agg128(xt, src64, dst64, zeros128)
    tt = _fused_call(acc1, xt, dinv, W1, b1, W2)
    acc2 = _agg128(tt, src64, dst64, zeros128)
    out = _softmax_call(acc2, tt, dinv, b2)
    return out[:n]


# speedup vs baseline: 37.3600x; 1.0675x over previous
import functools

import jax
import jax.numpy as jnp
from jax import lax
from jax.experimental import pallas as pl
from jax.experimental.pallas import tpu as pltpu
from jax.experimental.pallas import tpu_sc as plsc

N_NODES_ = 10000
N_EDGES_ = 320000
NP = 10240
NCORES = 2
NSUB = 16
NTILES = NCORES * NSUB
EDGES_PER_TILE = 10240
EPAD = EDGES_PER_TILE * NTILES
BATCH = 128
NBATCH = EDGES_PER_TILE // BATCH
ROWS_PER_TILE = NP // NSUB

_MESH = plsc.VectorSubcoreMesh(core_axis_name="c", subcore_axis_name="s")


@functools.partial(
    pl.kernel,
    out_type=[jax.ShapeDtypeStruct((NP,), jnp.float32),
              jax.ShapeDtypeStruct((NP,), jnp.float32)],
    mesh=_MESH,
    scratch_types=[
        pltpu.VMEM_SHARED((NSUB, NP), jnp.float32),
        pltpu.VMEM((NBATCH, BATCH), jnp.int32),
        pltpu.VMEM((NP,), jnp.float32),
        pltpu.VMEM((ROWS_PER_TILE,), jnp.float32),
        pltpu.VMEM((ROWS_PER_TILE,), jnp.float32),
    ],
    compiler_params=pltpu.CompilerParams(needs_layout_passes=False),
)
def _deg_kernel(dst_hbm, out0, out1, sh, dst_v, hist, red_a, red_b):
    c = lax.axis_index("c")
    s = lax.axis_index("s")
    wid = c * NSUB + s
    pltpu.sync_copy(dst_hbm.at[pl.ds(wid * NBATCH, NBATCH)], dst_v)

    zero16 = jnp.zeros((16,), jnp.float32)
    one16 = jnp.ones((16,), jnp.float32)

    def zstep(i, carry):
        hist[pl.ds(i * 16, 16)] = zero16
        return carry

    lax.fori_loop(0, NP // 16, zstep, 0)

    def hstep(j, carry):
        for k in range(BATCH // 16):
            idx = dst_v[j, pl.ds(k * 16, 16)]
            plsc.addupdate_scatter(hist, [idx], one16)
        return carry

    lax.fori_loop(0, NBATCH, hstep, 0)

    pltpu.sync_copy(hist, sh.at[s])
    plsc.subcore_barrier()

    sl = pl.ds(ROWS_PER_TILE * s, ROWS_PER_TILE)
    pltpu.sync_copy(sh.at[0].at[sl], red_a)
    for k in range(1, NSUB):
        pltpu.sync_copy(sh.at[k].at[sl], red_b)

        def astep(m, carry):
            red_a[pl.ds(m * 16, 16)] = (red_a[pl.ds(m * 16, 16)]
                                        + red_b[pl.ds(m * 16, 16)])
            return carry

        lax.fori_loop(0, ROWS_PER_TILE // 16, astep, 0)

    @pl.when(c == 0)
    def _():
        pltpu.sync_copy(red_a, out0.at[sl])

    @pl.when(c == 1)
    def _():
        pltpu.sync_copy(red_a, out1.at[sl])


SBATCH = 64
NBTOT = EPAD // SBATCH
NST0, NST1 = 5, 5
ST = 32
NBUF = 4
assert (NST0 + NST1) * ST * NSUB == NBTOT


def _make_agg(feat):
    @functools.partial(
        pl.kernel,
        out_type=jax.ShapeDtypeStruct((NCORES, NP, feat), jnp.float32),
        mesh=_MESH,
        scratch_types=[
            pltpu.VMEM_SHARED((NP, feat), jnp.float32),
            pltpu.VMEM((ST, SBATCH), jnp.int32),
            pltpu.VMEM((ST, SBATCH), jnp.int32),
            [pltpu.VMEM((SBATCH, feat), jnp.float32) for _ in range(NBUF)],
            [pltpu.SemaphoreType.DMA for _ in range(NBUF)],
        ],
    )
    def agg(x_hbm, src_hbm, dst_hbm, out_hbm,
            acc, src_v, dst_v, bufs, sems):
        c = lax.axis_index("c")
        s = lax.axis_index("s")

        zero16 = jnp.zeros((16,), jnp.float32)

        def zfill(i, carry):
            r = i // (feat // 16)
            k = i % (feat // 16)
            bufs[0][r, pl.ds(k * 16, 16)] = zero16
            return carry

        lax.fori_loop(0, SBATCH * feat // 16, zfill, 0)
        for r in range(ROWS_PER_TILE // SBATCH):
            pltpu.sync_copy(
                bufs[0],
                acc.at[pl.ds(ROWS_PER_TILE * s + r * SBATCH, SBATCH)])
        plsc.subcore_barrier()

        nst = jnp.where(c == 0, NST0, NST1)
        row0 = jnp.where(c == 0, ST * NST0 * s,
                         ST * NST0 * NSUB + ST * NST1 * s)

        def stage(h, carry):
            base = row0 + h * ST
            pltpu.sync_copy(src_hbm.at[pl.ds(base, ST)], src_v)
            pltpu.sync_copy(dst_hbm.at[pl.ds(base, ST)], dst_v)
            for q in range(NBUF - 1):
                pltpu.async_copy(x_hbm.at[src_v.at[q]], bufs[q], sems[q])

            def step(i, carry2):
                for q in range(NBUF):
                    j = NBUF * i + q
                    pltpu.make_async_copy(
                        x_hbm.at[src_v.at[j]], bufs[q], sems[q]).wait()
                    pltpu.sync_copy(bufs[q], acc.at[dst_v.at[j]], add=True)
                    qn = (q + NBUF - 1) % NBUF

                    @pl.when(j + NBUF - 1 < ST)
                    def _():
                        pltpu.async_copy(
                            x_hbm.at[src_v.at[j + NBUF - 1]],
                            bufs[qn], sems[qn])
                return carry2

            lax.fori_loop(0, ST // NBUF, step, 0)
            return carry

        lax.fori_loop(0, nst, stage, 0)
        plsc.subcore_barrier()
        sl = pl.ds(ROWS_PER_TILE * s, ROWS_PER_TILE)
        pltpu.sync_copy(acc.at[sl], out_hbm.at[c].at[sl])

    return agg


_agg128 = _make_agg(128)


BLK = 1024
GRID = NP // BLK


def _scale_body(deg0_ref, deg1_ref, x_ref, dinv_ref, xt_ref):
    d = deg0_ref[...] + deg1_ref[...] + 1.0
    di = lax.rsqrt(d)
    dinv_ref[...] = di
    xt_ref[...] = x_ref[...] * di


def _scale_call(deg0, deg1, x_pad):
    return pl.pallas_call(
        _scale_body,
        grid=(GRID,),
        in_specs=[
            pl.BlockSpec((BLK, 1), lambda i: (i, 0)),
            pl.BlockSpec((BLK, 1), lambda i: (i, 0)),
            pl.BlockSpec((BLK, 128), lambda i: (i, 0)),
        ],
        out_specs=[
            pl.BlockSpec((BLK, 1), lambda i: (i, 0)),
            pl.BlockSpec((BLK, 128), lambda i: (i, 0)),
        ],
        out_shape=[
            jax.ShapeDtypeStruct((NP, 1), jnp.float32),
            jax.ShapeDtypeStruct((NP, 128), jnp.float32),
        ],
    )(deg0, deg1, x_pad)


def _fused_body(acc_ref, xt_ref, dinv_ref, w1_ref, b1_ref, w2_ref, out_ref):
    di = dinv_ref[...]
    z = (acc_ref[0] + acc_ref[1] + xt_ref[...]) * di
    h = jnp.dot(z, w1_ref[...], preferred_element_type=jnp.float32)
    h = jnp.maximum(h + b1_ref[...], 0.0)
    t = jnp.dot(h, w2_ref[...], preferred_element_type=jnp.float32)
    out_ref[...] = jnp.concatenate(
        [t * di, jnp.zeros((t.shape[0], 64), jnp.float32)], axis=1)


def _fused_call(acc1, xt, dinv, W1, b1, W2):
    return pl.pallas_call(
        _fused_body,
        grid=(GRID,),
        in_specs=[
            pl.BlockSpec((NCORES, BLK, 128), lambda i: (0, i, 0)),
            pl.BlockSpec((BLK, 128), lambda i: (i, 0)),
            pl.BlockSpec((BLK, 1), lambda i: (i, 0)),
            pl.BlockSpec((128, 1024), lambda i: (0, 0)),
            pl.BlockSpec((1, 1024), lambda i: (0, 0)),
            pl.BlockSpec((1024, 64), lambda i: (0, 0)),
        ],
        out_specs=pl.BlockSpec((BLK, 128), lambda i: (i, 0)),
        out_shape=jax.ShapeDtypeStruct((NP, 128), jnp.float32),
    )(acc1, xt, dinv, W1, b1.reshape(1, 1024), W2)


def _softmax_body(acc_ref, tt_ref, dinv_ref, b2_ref, out_ref):
    z128 = (acc_ref[0] + acc_ref[1] + tt_ref[...]) * dinv_ref[...]
    z = z128[:, :64] + b2_ref[...]
    m = jnp.max(z, axis=1, keepdims=True)
    e = jnp.exp(z - m)
    out_ref[...] = e / jnp.sum(e, axis=1, keepdims=True)


def _softmax_call(acc2, tt, dinv, b2):
    return pl.pallas_call(
        _softmax_body,
        grid=(GRID,),
        in_specs=[
            pl.BlockSpec((NCORES, BLK, 128), lambda i: (0, i, 0)),
            pl.BlockSpec((BLK, 128), lambda i: (i, 0)),
            pl.BlockSpec((BLK, 1), lambda i: (i, 0)),
            pl.BlockSpec((1, 64), lambda i: (0, 0)),
        ],
        out_specs=pl.BlockSpec((BLK, 64), lambda i: (i, 0)),
        out_shape=jax.ShapeDtypeStruct((NP, 64), jnp.float32),
    )(acc2, tt, dinv, b2.reshape(1, 64))


def kernel(x, edge_index, W1, b1, W2, b2):
    n = x.shape[0]
    e = edge_index.shape[1]
    src = edge_index[0].astype(jnp.int32)
    dst = edge_index[1].astype(jnp.int32)
    pad = EPAD - e
    pad_idx = n + (jnp.arange(pad, dtype=jnp.int32) % (NP - n))
    src_p = jnp.concatenate([src, pad_idx])
    dst_p = jnp.concatenate([dst, pad_idx])
    src2d = src_p.reshape(EPAD // BATCH, BATCH)
    dst2d = dst_p.reshape(EPAD // BATCH, BATCH)
    src64 = src_p.reshape(EPAD // SBATCH, SBATCH)
    dst64 = dst_p.reshape(EPAD // SBATCH, SBATCH)
    x_pad = jnp.zeros((NP, 128), jnp.float32).at[:n].set(x)

    deg0, deg1 = _deg_kernel(dst2d)
    dinv, xt = _scale_call(deg0.reshape(NP, 1), deg1.reshape(NP, 1), x_pad)
    acc1 = _agg128(xt, src64, dst64)
    tt = _fused_call(acc1, xt, dinv, W1, b1, W2)
    acc2 = _agg128(tt, src64, dst64)
    out = _softmax_call(acc2, tt, dinv, b2)
    return out[:n]
